# baseline (speedup 1.0000x reference)
import jax
import jax.numpy as jnp
from jax.experimental import pallas as pl
from jax.experimental.pallas import tpu as pltpu

_NUM_CODES = 1024
_EMBED_DIM = 256
_BETA = 0.25
_BR = 512


def _vq_block_kernel(z_ref, ct_ref, c_ref, qt_ref, idx_ref, dsum_ref, cs_ref):
    c = c_ref[...]
    @pl.when(pl.program_id(0) == 0)
    def _():
        cs_ref[...] = jnp.sum(c * c, axis=1)[None, :]
    zb = z_ref[...]
    ct = ct_ref[...]
    mm = jnp.dot(zb, ct, preferred_element_type=jnp.float32)
    zs = jnp.sum(zb * zb, axis=1, keepdims=True)
    cs = cs_ref[...]
    d2 = jnp.maximum(zs + cs - 2.0 * mm, 0.0)
    dist = jnp.sqrt(d2)
    m = jnp.min(dist, axis=1, keepdims=True)
    iota = jax.lax.broadcasted_iota(jnp.int32, dist.shape, 1)
    idx = jnp.min(jnp.where(dist == m, iota, _NUM_CODES), axis=1)
    onehot = (iota == idx[:, None]).astype(jnp.bfloat16)
    qt = jax.lax.dot_general(
        c.astype(jnp.bfloat16), onehot, (((0,), (1,)), ((), ())),
        preferred_element_type=jnp.float32)
    qt_ref[...] = qt[None]
    idx_ref[...] = idx.reshape(1, 1, _BR)
    s = jnp.sum(m * m, keepdims=True).reshape(1, 1, 1)
    dsum_ref[...] = jnp.broadcast_to(s, (1, 1, 128))


def kernel(z, codebook):
    B, D, H, W = z.shape
    n = B * H * W
    nblk = n // _BR
    per_b = (H * W) // _BR
    z_flat = jnp.transpose(z, (0, 2, 3, 1)).reshape(-1, D)
    ct = codebook.T
    qt3, idx3, dsum = pl.pallas_call(
        _vq_block_kernel,
        grid=(nblk,),
        in_specs=[
            pl.BlockSpec((_BR, D), lambda i: (i, 0)),
            pl.BlockSpec((D, _NUM_CODES), lambda i: (0, 0)),
            pl.BlockSpec((_NUM_CODES, D), lambda i: (0, 0)),
        ],
        out_specs=[
            pl.BlockSpec((1, D, _BR), lambda i: (i // per_b, 0, i % per_b)),
            pl.BlockSpec((1, 1, _BR), lambda i: (i, 0, 0)),
            pl.BlockSpec((1, 1, 128), lambda i: (i, 0, 0)),
        ],
        out_shape=[
            jax.ShapeDtypeStruct((B, D, H * W), jnp.float32),
            jax.ShapeDtypeStruct((nblk, 1, _BR), jnp.int32),
            jax.ShapeDtypeStruct((nblk, 1, 128), jnp.float32),
        ],
        scratch_shapes=[pltpu.VMEM((1, _NUM_CODES), jnp.float32)],
    )(z_flat, ct, codebook)
    z_q = qt3.reshape(B, D, H, W)
    vq_loss = (1.0 + _BETA) * (jnp.sum(dsum[:, 0, 0]) / (n * D))
    indices = idx3.reshape(B, H, W)
    return (z_q, vq_loss, indices)
